# baseline (speedup 1.0000x reference)
import functools

import jax
import jax.numpy as jnp
from jax import lax
from jax.experimental import pallas as pl
from jax.experimental.pallas import tpu as pltpu
from jax.experimental.pallas import tpu_sc as plsc

N = 10000
E = 160000
D_IN = 300
D_HID = 128
D_OUT = 64

NP = 10240
B = 128
EP = 163840
ROWS = EP // B
NW = 32
RPW = ROWS // NW
TILES = 16
RPT = ROWS // TILES
NPT = NP // TILES
PAD = N
CH = 160
NBUF = 2
SLACK = 1
ROWSP = ROWS
RBLK = 640

_MESH = dict(core_axis_name="c", subcore_axis_name="s")


@functools.partial(
    pl.kernel,
    out_type=jax.ShapeDtypeStruct((2, NP, B), jnp.float32),
    mesh=plsc.VectorSubcoreMesh(**_MESH),
    scratch_types=[
        pltpu.VMEM((RPW, B), jnp.int32),
        pltpu.VMEM((B, B), jnp.float32),
        pltpu.VMEM((CH, B), jnp.float32),
        pltpu.VMEM_SHARED((NP, B), jnp.float32),
        pltpu.SemaphoreType.DMA,
    ],
)
def _deg_kernel(dst_hbm, deg_hbm, dst_v, ones_v, stage_v, acc_sh, dsem):
    c = lax.axis_index("c")
    s = lax.axis_index("s")
    wid = s * 2 + c
    zero16 = jnp.zeros((16,), jnp.float32)
    one16 = jnp.ones((16,), jnp.float32)

    def fill(i, carry):
        ones_v[i // 8, pl.ds((i % 8) * 16, 16)] = one16
        return carry

    lax.fori_loop(0, B * 8, fill, 0)

    def zfill(i, carry):
        stage_v[i // 8, pl.ds((i % 8) * 16, 16)] = zero16
        return carry

    lax.fori_loop(0, CH * 8, zfill, 0)

    def zinit(k, carry):
        pltpu.sync_copy(stage_v, acc_sh.at[pl.ds(s * NPT + k * CH, CH)])
        return carry

    lax.fori_loop(0, NPT // CH, zinit, 0)
    pltpu.sync_copy(dst_hbm.at[pl.ds(wid * RPW, RPW)], dst_v)
    plsc.subcore_barrier()

    def body(b, carry):
        pltpu.async_copy(ones_v, acc_sh.at[dst_v.at[b]], dsem, add=True)
        return carry

    lax.fori_loop(0, RPW, body, 0)

    def drain(b, carry):
        pltpu.make_async_copy(ones_v, acc_sh.at[dst_v.at[b]], dsem).wait()
        return carry

    lax.fori_loop(0, RPW, drain, 0)
    plsc.subcore_barrier()

    pltpu.sync_copy(acc_sh.at[pl.ds(s * NPT, NPT)],
                    deg_hbm.at[c].at[pl.ds(s * NPT, NPT)])


def _make_agg(d):
    @functools.partial(
        pl.kernel,
        out_type=jax.ShapeDtypeStruct((NP, d), jnp.float32),
        mesh=plsc.VectorSubcoreMesh(**_MESH),
        scratch_types=[
            pltpu.VMEM((RPW, B), jnp.int32),
            pltpu.VMEM((RPW, B), jnp.int32),
            pltpu.VMEM((NBUF, B, d), jnp.float32),
            pltpu.VMEM_SHARED((NP, d), jnp.float32),
            pltpu.SemaphoreType.DMA,
            pltpu.SemaphoreType.DMA,
        ],
    )
    def agg(src_hbm, dst_hbm, y_hbm, out_hbm, src_v, dst_v, rows_v,
            acc_sh, gsem, ssem):
        c = lax.axis_index("c")
        s = lax.axis_index("s")

        @pl.when(c == 0)
        def _():
            pltpu.sync_copy(y_hbm.at[pl.ds(s * NPT, NPT)],
                            acc_sh.at[pl.ds(s * NPT, NPT)])

        plsc.subcore_barrier()

        @pl.when(c == 0)
        def _():
            def phase(ph, carry):
                base = s * 2 * RPW + ph * RPW
                pltpu.sync_copy(src_hbm.at[pl.ds(base, RPW)], src_v)
                pltpu.sync_copy(dst_hbm.at[pl.ds(base, RPW)], dst_v)

                def prefire(k, carry):
                    pltpu.async_copy(y_hbm.at[src_v.at[k]], rows_v.at[k],
                                     gsem)
                    return carry

                lax.fori_loop(0, NBUF, prefire, 0)

                def body(b, carry):
                    buf = b % NBUF
                    pltpu.make_async_copy(y_hbm.at[src_v.at[b]],
                                          rows_v.at[buf], gsem).wait()
                    pltpu.async_copy(rows_v.at[buf], acc_sh.at[dst_v.at[b]],
                                     ssem, add=True)

                    @pl.when(b >= SLACK)
                    def _():
                        bb = b - SLACK
                        pltpu.make_async_copy(rows_v.at[bb % NBUF],
                                              acc_sh.at[dst_v.at[bb]],
                                              ssem).wait()

                        @pl.when(bb + NBUF < RPW)
                        def _():
                            g = bb + NBUF
                            pltpu.async_copy(y_hbm.at[src_v.at[g]],
                                             rows_v.at[g % NBUF], gsem)

                    return carry

                lax.fori_loop(0, RPW, body, 0)

                def sdrain(i, carry):
                    bb = RPW - SLACK + i
                    pltpu.make_async_copy(rows_v.at[bb % NBUF],
                                          acc_sh.at[dst_v.at[bb]],
                                          ssem).wait()
                    return carry

                lax.fori_loop(0, SLACK, sdrain, 0)
                return carry

            lax.fori_loop(0, 2, phase, 0)

        plsc.subcore_barrier()

        @pl.when(c == 0)
        def _():
            pltpu.sync_copy(acc_sh.at[pl.ds(s * NPT, NPT)],
                            out_hbm.at[pl.ds(s * NPT, NPT)])

    return agg


_agg_hid = _make_agg(D_HID)


def _mm1_body(q0_ref, q1_ref, x_ref, w_ref, y_ref, dis_ref):
    deg = (q0_ref[...] + q1_ref[...])[:, :1]
    dis = lax.rsqrt(deg + 1.0)
    xw = jnp.dot(x_ref[...], w_ref[...], preferred_element_type=jnp.float32)
    y_ref[...] = dis * xw
    dis_ref[...] = dis


def _mm2_body(p_ref, dis_ref, b1_ref, w2_ref, y2_ref):
    dis = dis_ref[...]
    h = jnp.maximum(dis * p_ref[...] + b1_ref[...], 0.0)
    y2_ref[...] = dis * jnp.dot(h, w2_ref[...],
                                preferred_element_type=jnp.float32)


def _fin_body(q_ref, dis_ref, b2_ref, out_ref):
    out_ref[...] = (dis_ref[...] * q_ref[...][:, :D_OUT]) + b2_ref[...]


def _row_spec(d):
    return pl.BlockSpec((RBLK, d), lambda i: (i, 0))


def _full_spec(r, d):
    return pl.BlockSpec((r, d), lambda i: (0, 0))


_GRID = NP // RBLK

_mm1 = pl.pallas_call(
    _mm1_body,
    grid=(_GRID,),
    in_specs=[_row_spec(B), _row_spec(B), _row_spec(D_IN),
              _full_spec(D_IN, D_HID)],
    out_specs=[_row_spec(D_HID), _row_spec(1)],
    out_shape=[jax.ShapeDtypeStruct((NP, D_HID), jnp.float32),
               jax.ShapeDtypeStruct((NP, 1), jnp.float32)],
)

_mm2 = pl.pallas_call(
    _mm2_body,
    grid=(_GRID,),
    in_specs=[_row_spec(D_HID), _row_spec(1), _full_spec(1, D_HID),
              _full_spec(D_HID, D_HID)],
    out_specs=_row_spec(D_HID),
    out_shape=jax.ShapeDtypeStruct((NP, D_HID), jnp.float32),
)

_fin = pl.pallas_call(
    _fin_body,
    grid=(_GRID,),
    in_specs=[_row_spec(D_HID), _row_spec(1), _full_spec(1, D_OUT)],
    out_specs=_row_spec(D_OUT),
    out_shape=jax.ShapeDtypeStruct((NP, D_OUT), jnp.float32),
)


@jax.jit
def _run(x, edge_index, W1, b1, W2, b2):
    pad = jnp.full((ROWSP * B - E,), PAD, jnp.int32)
    srcp = jnp.concatenate([edge_index[0], pad]).reshape(ROWSP, B)
    dstp = jnp.concatenate([edge_index[1], pad]).reshape(ROWSP, B)
    xp = jnp.pad(x, ((0, NP - N), (0, 0)))

    deg = _deg_kernel(dstp)
    y1, dis = _mm1(deg[0], deg[1], xp, W1)
    p = _agg_hid(srcp, dstp, y1)
    W2p = jnp.pad(W2, ((0, 0), (0, D_HID - D_OUT)))
    y2 = _mm2(p, dis, b1.reshape(1, D_HID), W2p)
    q = _agg_hid(srcp, dstp, y2)
    out = _fin(q, dis, b2.reshape(1, D_OUT))
    return out[:N]


def kernel(x, edge_index, W1, b1, W2, b2):
    return _run(x, edge_index, W1, b1, W2, b2)


# speedup vs baseline: 1.1942x; 1.1942x over previous
import functools

import jax
import jax.numpy as jnp
from jax import lax
from jax.experimental import pallas as pl
from jax.experimental.pallas import tpu as pltpu
from jax.experimental.pallas import tpu_sc as plsc

N = 10000
E = 160000
D_IN = 300
D_HID = 128
D_OUT = 64

NP = 10240
B = 128
EP = 163840
ROWS = EP // B
NW = 32
RPW = ROWS // NW
TILES = 16
RPT = ROWS // TILES
NPT = NP // TILES
PAD = N
CH = 160
NBUF = 2
SLACK = 1
CS0 = 64
CS1 = 2 * RPW - CS0
ROWSP = ROWS
RBLK = 640

_MESH = dict(core_axis_name="c", subcore_axis_name="s")


@functools.partial(
    pl.kernel,
    out_type=jax.ShapeDtypeStruct((2, NP, B), jnp.float32),
    mesh=plsc.VectorSubcoreMesh(**_MESH),
    scratch_types=[
        pltpu.VMEM((RPW, B), jnp.int32),
        pltpu.VMEM((B, B), jnp.float32),
        pltpu.VMEM((CH, B), jnp.float32),
        pltpu.VMEM_SHARED((NP, B), jnp.float32),
        pltpu.SemaphoreType.DMA,
    ],
)
def _deg_kernel(dst_hbm, deg_hbm, dst_v, ones_v, stage_v, acc_sh, dsem):
    c = lax.axis_index("c")
    s = lax.axis_index("s")
    wid = s * 2 + c
    zero16 = jnp.zeros((16,), jnp.float32)
    one16 = jnp.ones((16,), jnp.float32)

    def fill(i, carry):
        ones_v[i // 8, pl.ds((i % 8) * 16, 16)] = one16
        return carry

    lax.fori_loop(0, B * 8, fill, 0)

    def zfill(i, carry):
        stage_v[i // 8, pl.ds((i % 8) * 16, 16)] = zero16
        return carry

    lax.fori_loop(0, CH * 8, zfill, 0)

    def zinit(k, carry):
        pltpu.sync_copy(stage_v, acc_sh.at[pl.ds(s * NPT + k * CH, CH)])
        return carry

    lax.fori_loop(0, NPT // CH, zinit, 0)
    pltpu.sync_copy(dst_hbm.at[pl.ds(wid * RPW, RPW)], dst_v)
    plsc.subcore_barrier()

    def body(b, carry):
        pltpu.async_copy(ones_v, acc_sh.at[dst_v.at[b]], dsem, add=True)
        return carry

    lax.fori_loop(0, RPW, body, 0)

    def drain(b, carry):
        pltpu.make_async_copy(ones_v, acc_sh.at[dst_v.at[b]], dsem).wait()
        return carry

    lax.fori_loop(0, RPW, drain, 0)
    plsc.subcore_barrier()

    pltpu.sync_copy(acc_sh.at[pl.ds(s * NPT, NPT)],
                    deg_hbm.at[c].at[pl.ds(s * NPT, NPT)])


def _make_agg(d):
    @functools.partial(
        pl.kernel,
        out_type=jax.ShapeDtypeStruct((2, NP, d), jnp.float32),
        mesh=plsc.VectorSubcoreMesh(**_MESH),
        scratch_types=[
            pltpu.VMEM((CS0, B), jnp.int32),
            pltpu.VMEM((CS0, B), jnp.int32),
            pltpu.VMEM((NBUF, B, d), jnp.float32),
            pltpu.VMEM_SHARED((NP, d), jnp.float32),
            pltpu.SemaphoreType.DMA,
            pltpu.SemaphoreType.DMA,
        ],
    )
    def agg(src_hbm, dst_hbm, y_hbm, out_hbm, src_v, dst_v, rows_v,
            acc_sh, gsem, ssem):
        c = lax.axis_index("c")
        s = lax.axis_index("s")
        nb = jnp.where(c == 0, CS0, CS1)
        base = jnp.where(c == 0, s * CS0, TILES * CS0 + s * CS1)
        pltpu.sync_copy(y_hbm.at[pl.ds(s * NPT, NPT)],
                        acc_sh.at[pl.ds(s * NPT, NPT)])
        pltpu.sync_copy(src_hbm.at[pl.ds(base, CS0)], src_v)
        pltpu.sync_copy(dst_hbm.at[pl.ds(base, CS0)], dst_v)
        plsc.subcore_barrier()

        def prefire(k, carry):
            pltpu.async_copy(y_hbm.at[src_v.at[k]], rows_v.at[k], gsem)
            return carry

        lax.fori_loop(0, NBUF, prefire, 0)

        def body(b, carry):
            buf = b % NBUF
            pltpu.make_async_copy(y_hbm.at[src_v.at[b]], rows_v.at[buf],
                                  gsem).wait()
            pltpu.async_copy(rows_v.at[buf], acc_sh.at[dst_v.at[b]], ssem,
                             add=True)

            @pl.when(b >= SLACK)
            def _():
                bb = b - SLACK
                pltpu.make_async_copy(rows_v.at[bb % NBUF],
                                      acc_sh.at[dst_v.at[bb]], ssem).wait()

                @pl.when(bb + NBUF < nb)
                def _():
                    g = bb + NBUF
                    pltpu.async_copy(y_hbm.at[src_v.at[g]],
                                     rows_v.at[g % NBUF], gsem)

            return carry

        lax.fori_loop(0, nb, body, 0)

        def sdrain(i, carry):
            bb = nb - SLACK + i
            pltpu.make_async_copy(rows_v.at[bb % NBUF],
                                  acc_sh.at[dst_v.at[bb]], ssem).wait()
            return carry

        lax.fori_loop(0, SLACK, sdrain, 0)
        plsc.subcore_barrier()
        pltpu.sync_copy(acc_sh.at[pl.ds(s * NPT, NPT)],
                        out_hbm.at[c].at[pl.ds(s * NPT, NPT)])

    return agg


_agg_hid = _make_agg(D_HID)


def _mmraw_body(x_ref, w_ref, xw_ref):
    xw_ref[...] = jnp.dot(x_ref[...], w_ref[...],
                          preferred_element_type=jnp.float32)


def _scale_body(q0_ref, q1_ref, xw_ref, y_ref, dis_ref):
    deg = (q0_ref[...] + q1_ref[...])[:, :1]
    dis = lax.rsqrt(deg + 1.0)
    y_ref[...] = dis * xw_ref[...]
    dis_ref[...] = dis


def _mm2_body(p0_ref, p1_ref, y1_ref, dis_ref, b1_ref, w2_ref, y2_ref):
    dis = dis_ref[...]
    agg = p0_ref[...] + p1_ref[...] - y1_ref[...]
    h = jnp.maximum(dis * agg + b1_ref[...], 0.0)
    y2_ref[...] = dis * jnp.dot(h, w2_ref[...],
                                preferred_element_type=jnp.float32)


def _fin_body(q0_ref, q1_ref, y2_ref, dis_ref, b2_ref, out_ref):
    agg = (q0_ref[...] + q1_ref[...] - y2_ref[...])[:, :D_OUT]
    out_ref[...] = dis_ref[...] * agg + b2_ref[...]


def _row_spec(d):
    return pl.BlockSpec((RBLK, d), lambda i: (i, 0))


def _full_spec(r, d):
    return pl.BlockSpec((r, d), lambda i: (0, 0))


_GRID = NP // RBLK

_mmraw = pl.pallas_call(
    _mmraw_body,
    grid=(_GRID,),
    in_specs=[_row_spec(D_IN), _full_spec(D_IN, D_HID)],
    out_specs=_row_spec(D_HID),
    out_shape=jax.ShapeDtypeStruct((NP, D_HID), jnp.float32),
)

_scale = pl.pallas_call(
    _scale_body,
    grid=(_GRID,),
    in_specs=[_row_spec(B), _row_spec(B), _row_spec(D_HID)],
    out_specs=[_row_spec(D_HID), _row_spec(1)],
    out_shape=[jax.ShapeDtypeStruct((NP, D_HID), jnp.float32),
               jax.ShapeDtypeStruct((NP, 1), jnp.float32)],
)

_mm2 = pl.pallas_call(
    _mm2_body,
    grid=(_GRID,),
    in_specs=[_row_spec(D_HID), _row_spec(D_HID), _row_spec(D_HID),
              _row_spec(1), _full_spec(1, D_HID), _full_spec(D_HID, D_HID)],
    out_specs=_row_spec(D_HID),
    out_shape=jax.ShapeDtypeStruct((NP, D_HID), jnp.float32),
)

_fin = pl.pallas_call(
    _fin_body,
    grid=(_GRID,),
    in_specs=[_row_spec(D_HID), _row_spec(D_HID), _row_spec(D_HID),
              _row_spec(1), _full_spec(1, D_OUT)],
    out_specs=_row_spec(D_OUT),
    out_shape=jax.ShapeDtypeStruct((NP, D_OUT), jnp.float32),
)


@jax.jit
def _run(x, edge_index, W1, b1, W2, b2):
    ep = jnp.pad(edge_index, ((0, 0), (0, ROWSP * B - E)),
                 constant_values=PAD)
    srcp = ep[0].reshape(ROWSP, B)
    dstp = ep[1].reshape(ROWSP, B)
    xp = jnp.pad(x, ((0, NP - N), (0, 0)))

    deg = _deg_kernel(dstp)
    xw1 = _mmraw(xp, W1)
    y1, dis = _scale(deg[0], deg[1], xw1)
    p = _agg_hid(srcp, dstp, y1)
    W2p = jnp.pad(W2, ((0, 0), (0, D_HID - D_OUT)))
    y2 = _mm2(p[0], p[1], y1, dis, b1.reshape(1, D_HID), W2p)
    q = _agg_hid(srcp, dstp, y2)
    out = _fin(q[0], q[1], y2, dis, b2.reshape(1, D_OUT))
    return out[:N]


def kernel(x, edge_index, W1, b1, W2, b2):
    return _run(x, edge_index, W1, b1, W2, b2)


# speedup vs baseline: 1.1968x; 1.0022x over previous
import functools

import jax
import jax.numpy as jnp
from jax import lax
from jax.experimental import pallas as pl
from jax.experimental.pallas import tpu as pltpu
from jax.experimental.pallas import tpu_sc as plsc

N = 10000
E = 160000
D_IN = 300
D_HID = 128
D_OUT = 64

NP = 10240
B = 128
EP = 163840
ROWS = EP // B
NW = 32
RPW = ROWS // NW
TILES = 16
RPT = ROWS // TILES
NPT = NP // TILES
PAD = N
CH = 160
NBUF = 2
SLACK = 1
CS0 = 64
CS1 = 2 * RPW - CS0
ROWS0 = E // B
ROWSP = ROWS + 64
RBLK = 640

_MESH = dict(core_axis_name="c", subcore_axis_name="s")


@functools.partial(
    pl.kernel,
    out_type=jax.ShapeDtypeStruct((2, NP, B), jnp.float32),
    mesh=plsc.VectorSubcoreMesh(**_MESH),
    scratch_types=[
        pltpu.VMEM((RPW, B), jnp.int32),
        pltpu.VMEM((B, B), jnp.float32),
        pltpu.VMEM((CH, B), jnp.float32),
        pltpu.VMEM_SHARED((NP, B), jnp.float32),
        pltpu.SemaphoreType.DMA,
    ],
)
def _deg_kernel(dst_hbm, deg_hbm, dst_v, ones_v, stage_v, acc_sh, dsem):
    c = lax.axis_index("c")
    s = lax.axis_index("s")
    wid = s * 2 + c
    zero16 = jnp.zeros((16,), jnp.float32)
    one16 = jnp.ones((16,), jnp.float32)

    def fill(i, carry):
        ones_v[i // 8, pl.ds((i % 8) * 16, 16)] = one16
        return carry

    lax.fori_loop(0, B * 8, fill, 0)

    def zfill(i, carry):
        stage_v[i // 8, pl.ds((i % 8) * 16, 16)] = zero16
        return carry

    lax.fori_loop(0, CH * 8, zfill, 0)

    def zinit(k, carry):
        pltpu.sync_copy(stage_v, acc_sh.at[pl.ds(s * NPT + k * CH, CH)])
        return carry

    lax.fori_loop(0, NPT // CH, zinit, 0)
    pltpu.sync_copy(dst_hbm.at[pl.ds(wid * RPW, RPW)], dst_v)
    plsc.subcore_barrier()

    def body(b, carry):
        pltpu.async_copy(ones_v, acc_sh.at[dst_v.at[b]], dsem, add=True)
        return carry

    lax.fori_loop(0, RPW, body, 0)

    def drain(b, carry):
        pltpu.make_async_copy(ones_v, acc_sh.at[dst_v.at[b]], dsem).wait()
        return carry

    lax.fori_loop(0, RPW, drain, 0)
    plsc.subcore_barrier()

    pltpu.sync_copy(acc_sh.at[pl.ds(s * NPT, NPT)],
                    deg_hbm.at[c].at[pl.ds(s * NPT, NPT)])


def _make_agg(d):
    @functools.partial(
        pl.kernel,
        out_type=jax.ShapeDtypeStruct((2, NP, d), jnp.float32),
        mesh=plsc.VectorSubcoreMesh(**_MESH),
        scratch_types=[
            pltpu.VMEM((CS0, B), jnp.int32),
            pltpu.VMEM((CS0, B), jnp.int32),
            pltpu.VMEM((NBUF, B, d), jnp.float32),
            pltpu.VMEM_SHARED((NP, d), jnp.float32),
            pltpu.SemaphoreType.DMA,
            pltpu.SemaphoreType.DMA,
        ],
    )
    def agg(src_hbm, dst_hbm, y_hbm, out_hbm, src_v, dst_v, rows_v,
            acc_sh, gsem, ssem):
        c = lax.axis_index("c")
        s = lax.axis_index("s")
        nb = jnp.where(c == 0, CS0, CS1)
        base = jnp.where(c == 0, s * CS0, TILES * CS0 + s * CS1)

        @pl.when(c == 0)
        def _():
            pltpu.sync_copy(y_hbm.at[pl.ds(s * NPT, NPT)],
                            acc_sh.at[pl.ds(s * NPT, NPT)])

        @pl.when(c == 1)
        def _():
            zero16 = jnp.zeros((16,), jnp.float32)

            def zfill(i, carry):
                rows_v[0, i // (d // 16), pl.ds((i % (d // 16)) * 16, 16)] = (
                    zero16)
                return carry

            lax.fori_loop(0, B * (d // 16), zfill, 0)

            def zinit(k, carry):
                pltpu.sync_copy(rows_v.at[0],
                                acc_sh.at[pl.ds(s * NPT + k * B, B)])
                return carry

            lax.fori_loop(0, NPT // B, zinit, 0)

        pltpu.sync_copy(src_hbm.at[pl.ds(base, CS0)], src_v)
        pltpu.sync_copy(dst_hbm.at[pl.ds(base, CS0)], dst_v)
        plsc.subcore_barrier()

        def prefire(k, carry):
            pltpu.async_copy(y_hbm.at[src_v.at[k]], rows_v.at[k], gsem)
            return carry

        lax.fori_loop(0, NBUF, prefire, 0)

        def body(b, carry):
            buf = b % NBUF
            pltpu.make_async_copy(y_hbm.at[src_v.at[b]], rows_v.at[buf],
                                  gsem).wait()
            pltpu.async_copy(rows_v.at[buf], acc_sh.at[dst_v.at[b]], ssem,
                             add=True)

            @pl.when(b >= SLACK)
            def _():
                bb = b - SLACK
                pltpu.make_async_copy(rows_v.at[bb % NBUF],
                                      acc_sh.at[dst_v.at[bb]], ssem).wait()

                @pl.when(bb + NBUF < nb)
                def _():
                    g = bb + NBUF
                    pltpu.async_copy(y_hbm.at[src_v.at[g]],
                                     rows_v.at[g % NBUF], gsem)

            return carry

        lax.fori_loop(0, nb, body, 0)

        def sdrain(i, carry):
            bb = nb - SLACK + i
            pltpu.make_async_copy(rows_v.at[bb % NBUF],
                                  acc_sh.at[dst_v.at[bb]], ssem).wait()
            return carry

        lax.fori_loop(0, SLACK, sdrain, 0)
        plsc.subcore_barrier()
        pltpu.sync_copy(acc_sh.at[pl.ds(s * NPT, NPT)],
                        out_hbm.at[c].at[pl.ds(s * NPT, NPT)])

    return agg


_agg_hid = _make_agg(D_HID)


def _epad_body(s_ref, d_ref, sp_ref, dp_ref):
    tail = jnp.full((ROWSP - ROWS0, B), PAD, jnp.int32)
    sp_ref[...] = jnp.concatenate([s_ref[...], tail], axis=0)
    dp_ref[...] = jnp.concatenate([d_ref[...], tail], axis=0)


_epad = pl.pallas_call(
    _epad_body,
    out_shape=[jax.ShapeDtypeStruct((ROWSP, B), jnp.int32),
               jax.ShapeDtypeStruct((ROWSP, B), jnp.int32)],
)


def _mmraw_body(x_ref, w_ref, xw_ref):
    xw_ref[...] = jnp.dot(x_ref[...], w_ref[...],
                          preferred_element_type=jnp.float32)


def _scale_body(q0_ref, q1_ref, xw_ref, y_ref, dis_ref):
    deg = (q0_ref[...] + q1_ref[...])[:, :1]
    dis = lax.rsqrt(deg + 1.0)
    y_ref[...] = dis * xw_ref[...]
    dis_ref[...] = dis


def _mm2_body(p0_ref, p1_ref, dis_ref, b1_ref, w2_ref, y2_ref):
    dis = dis_ref[...]
    agg = p0_ref[...] + p1_ref[...]
    h = jnp.maximum(dis * agg + b1_ref[...], 0.0)
    y2_ref[...] = dis * jnp.dot(h, w2_ref[...],
                                preferred_element_type=jnp.float32)


def _fin_body(q0_ref, q1_ref, dis_ref, b2_ref, out_ref):
    agg = (q0_ref[...] + q1_ref[...])[:, :D_OUT]
    out_ref[...] = dis_ref[...] * agg + b2_ref[...]


def _row_spec(d):
    return pl.BlockSpec((RBLK, d), lambda i: (i, 0))


def _full_spec(r, d):
    return pl.BlockSpec((r, d), lambda i: (0, 0))


_GRID = NP // RBLK

_mmraw = pl.pallas_call(
    _mmraw_body,
    grid=(_GRID,),
    in_specs=[_row_spec(D_IN), _full_spec(D_IN, D_HID)],
    out_specs=_row_spec(D_HID),
    out_shape=jax.ShapeDtypeStruct((NP, D_HID), jnp.float32),
)

_scale = pl.pallas_call(
    _scale_body,
    grid=(_GRID,),
    in_specs=[_row_spec(B), _row_spec(B), _row_spec(D_HID)],
    out_specs=[_row_spec(D_HID), _row_spec(1)],
    out_shape=[jax.ShapeDtypeStruct((NP, D_HID), jnp.float32),
               jax.ShapeDtypeStruct((NP, 1), jnp.float32)],
)

_mm2 = pl.pallas_call(
    _mm2_body,
    grid=(_GRID,),
    in_specs=[_row_spec(D_HID), _row_spec(D_HID),
              _row_spec(1), _full_spec(1, D_HID), _full_spec(D_HID, D_HID)],
    out_specs=_row_spec(D_HID),
    out_shape=jax.ShapeDtypeStruct((NP, D_HID), jnp.float32),
)

_fin = pl.pallas_call(
    _fin_body,
    grid=(_GRID,),
    in_specs=[_row_spec(D_HID), _row_spec(D_HID),
              _row_spec(1), _full_spec(1, D_OUT)],
    out_specs=_row_spec(D_OUT),
    out_shape=jax.ShapeDtypeStruct((NP, D_OUT), jnp.float32),
)


@jax.jit
def _run(x, edge_index, W1, b1, W2, b2):
    srcp, dstp = _epad(edge_index[0].reshape(ROWS0, B),
                       edge_index[1].reshape(ROWS0, B))
    xp = jnp.pad(x, ((0, NP - N), (0, 0)))

    deg = _deg_kernel(dstp)
    xw1 = _mmraw(xp, W1)
    y1, dis = _scale(deg[0], deg[1], xw1)
    p = _agg_hid(srcp, dstp, y1)
    W2p = jnp.pad(W2, ((0, 0), (0, D_HID - D_OUT)))
    y2 = _mm2(p[0], p[1], dis, b1.reshape(1, D_HID), W2p)
    q = _agg_hid(srcp, dstp, y2)
    out = _fin(q[0], q[1], dis, b2.reshape(1, D_OUT))
    return out[:N]


def kernel(x, edge_index, W1, b1, W2, b2):
    return _run(x, edge_index, W1, b1, W2, b2)


# speedup vs baseline: 1.1972x; 1.0003x over previous
import functools

import jax
import jax.numpy as jnp
from jax import lax
from jax.experimental import pallas as pl
from jax.experimental.pallas import tpu as pltpu
from jax.experimental.pallas import tpu_sc as plsc

N = 10000
E = 160000
D_IN = 300
D_HID = 128
D_OUT = 64

NP = 10240
B = 128
EP = 163840
ROWS = EP // B
NW = 32
RPW = ROWS // NW
TILES = 16
RPT = ROWS // TILES
NPT = NP // TILES
PAD = N
CH = 160
NBUF = 2
SLACK = 1
CS0 = 64
CS1 = 2 * RPW - CS0
ROWS0 = E // B
ROWSP = ROWS + 64
RBLK = 640

_MESH = dict(core_axis_name="c", subcore_axis_name="s")


_SC_PARAMS = pltpu.CompilerParams(use_tc_tiling_on_sc=True)


@functools.partial(
    pl.kernel,
    out_type=jax.ShapeDtypeStruct((2, NP, B), jnp.float32),
    mesh=plsc.VectorSubcoreMesh(**_MESH),
    compiler_params=_SC_PARAMS,
    scratch_types=[
        pltpu.VMEM((RPW, B), jnp.int32),
        pltpu.VMEM((B, B), jnp.float32),
        pltpu.VMEM((CH, B), jnp.float32),
        pltpu.VMEM_SHARED((NP, B), jnp.float32),
        pltpu.SemaphoreType.DMA,
    ],
)
def _deg_kernel(dst_hbm, deg_hbm, dst_v, ones_v, stage_v, acc_sh, dsem):
    c = lax.axis_index("c")
    s = lax.axis_index("s")
    wid = s * 2 + c
    zero16 = jnp.zeros((16,), jnp.float32)
    one16 = jnp.ones((16,), jnp.float32)

    def fill(i, carry):
        ones_v[i // 8, pl.ds((i % 8) * 16, 16)] = one16
        return carry

    lax.fori_loop(0, B * 8, fill, 0)

    def zfill(i, carry):
        stage_v[i // 8, pl.ds((i % 8) * 16, 16)] = zero16
        return carry

    lax.fori_loop(0, CH * 8, zfill, 0)

    def zinit(k, carry):
        pltpu.sync_copy(stage_v, acc_sh.at[pl.ds(s * NPT + k * CH, CH)])
        return carry

    lax.fori_loop(0, NPT // CH, zinit, 0)
    pltpu.sync_copy(dst_hbm.at[pl.ds(wid * RPW, RPW)], dst_v)
    plsc.subcore_barrier()

    def body(b, carry):
        pltpu.async_copy(ones_v, acc_sh.at[dst_v.at[b]], dsem, add=True)
        return carry

    lax.fori_loop(0, RPW, body, 0)

    def drain(b, carry):
        pltpu.make_async_copy(ones_v, acc_sh.at[dst_v.at[b]], dsem).wait()
        return carry

    lax.fori_loop(0, RPW, drain, 0)
    plsc.subcore_barrier()

    pltpu.sync_copy(acc_sh.at[pl.ds(s * NPT, NPT)],
                    deg_hbm.at[c].at[pl.ds(s * NPT, NPT)])


def _make_agg(d):
    @functools.partial(
        pl.kernel,
        out_type=jax.ShapeDtypeStruct((2, NP, d), jnp.float32),
        mesh=plsc.VectorSubcoreMesh(**_MESH),
        compiler_params=_SC_PARAMS,
        scratch_types=[
            pltpu.VMEM((CS0, B), jnp.int32),
            pltpu.VMEM((CS0, B), jnp.int32),
            pltpu.VMEM((NBUF, B, d), jnp.float32),
            pltpu.VMEM_SHARED((NP, d), jnp.float32),
            pltpu.SemaphoreType.DMA,
            pltpu.SemaphoreType.DMA,
        ],
    )
    def agg(src_hbm, dst_hbm, y_hbm, out_hbm, src_v, dst_v, rows_v,
            acc_sh, gsem, ssem):
        c = lax.axis_index("c")
        s = lax.axis_index("s")
        nb = jnp.where(c == 0, CS0, CS1)
        base = jnp.where(c == 0, s * CS0, TILES * CS0 + s * CS1)

        @pl.when(c == 0)
        def _():
            pltpu.sync_copy(y_hbm.at[pl.ds(s * NPT, NPT)],
                            acc_sh.at[pl.ds(s * NPT, NPT)])

        @pl.when(c == 1)
        def _():
            zero16 = jnp.zeros((16,), jnp.float32)

            def zfill(i, carry):
                rows_v[0, i // (d // 16), pl.ds((i % (d // 16)) * 16, 16)] = (
                    zero16)
                return carry

            lax.fori_loop(0, B * (d // 16), zfill, 0)

            def zinit(k, carry):
                pltpu.sync_copy(rows_v.at[0],
                                acc_sh.at[pl.ds(s * NPT + k * B, B)])
                return carry

            lax.fori_loop(0, NPT // B, zinit, 0)

        pltpu.sync_copy(src_hbm.at[pl.ds(base, CS0)], src_v)
        pltpu.sync_copy(dst_hbm.at[pl.ds(base, CS0)], dst_v)
        plsc.subcore_barrier()

        def prefire(k, carry):
            pltpu.async_copy(y_hbm.at[src_v.at[k]], rows_v.at[k], gsem)
            return carry

        lax.fori_loop(0, NBUF, prefire, 0)

        def body(b, carry):
            buf = b % NBUF
            pltpu.make_async_copy(y_hbm.at[src_v.at[b]], rows_v.at[buf],
                                  gsem).wait()
            pltpu.async_copy(rows_v.at[buf], acc_sh.at[dst_v.at[b]], ssem,
                             add=True)

            @pl.when(b >= SLACK)
            def _():
                bb = b - SLACK
                pltpu.make_async_copy(rows_v.at[bb % NBUF],
                                      acc_sh.at[dst_v.at[bb]], ssem).wait()

                @pl.when(bb + NBUF < nb)
                def _():
                    g = bb + NBUF
                    pltpu.async_copy(y_hbm.at[src_v.at[g]],
                                     rows_v.at[g % NBUF], gsem)

            return carry

        lax.fori_loop(0, nb, body, 0)

        def sdrain(i, carry):
            bb = nb - SLACK + i
            pltpu.make_async_copy(rows_v.at[bb % NBUF],
                                  acc_sh.at[dst_v.at[bb]], ssem).wait()
            return carry

        lax.fori_loop(0, SLACK, sdrain, 0)
        plsc.subcore_barrier()
        pltpu.sync_copy(acc_sh.at[pl.ds(s * NPT, NPT)],
                        out_hbm.at[c].at[pl.ds(s * NPT, NPT)])

    return agg


_agg_hid = _make_agg(D_HID)


def _epad_body(s_ref, d_ref, sp_ref, dp_ref):
    tail = jnp.full((ROWSP - ROWS0, B), PAD, jnp.int32)
    sp_ref[...] = jnp.concatenate([s_ref[...], tail], axis=0)
    dp_ref[...] = jnp.concatenate([d_ref[...], tail], axis=0)


_epad = pl.pallas_call(
    _epad_body,
    out_shape=[jax.ShapeDtypeStruct((ROWSP, B), jnp.int32),
               jax.ShapeDtypeStruct((ROWSP, B), jnp.int32)],
)


def _mmraw_body(x_ref, w_ref, xw_ref):
    xw_ref[...] = jnp.dot(x_ref[...], w_ref[...],
                          preferred_element_type=jnp.float32)


def _scale_body(q0_ref, q1_ref, xw_ref, y_ref, dis_ref):
    deg = (q0_ref[...] + q1_ref[...])[:, :1]
    dis = lax.rsqrt(deg + 1.0)
    y_ref[...] = dis * xw_ref[...]
    dis_ref[...] = dis


def _mm2_body(p0_ref, p1_ref, dis_ref, b1_ref, w2_ref, y2_ref):
    dis = dis_ref[...]
    agg = p0_ref[...] + p1_ref[...]
    h = jnp.maximum(dis * agg + b1_ref[...], 0.0)
    y2_ref[...] = dis * jnp.dot(h, w2_ref[...],
                                preferred_element_type=jnp.float32)


def _fin_body(q0_ref, q1_ref, dis_ref, b2_ref, out_ref):
    agg = (q0_ref[...] + q1_ref[...])[:, :D_OUT]
    out_ref[...] = dis_ref[...] * agg + b2_ref[...]


def _row_spec(d):
    return pl.BlockSpec((RBLK, d), lambda i: (i, 0))


def _full_spec(r, d):
    return pl.BlockSpec((r, d), lambda i: (0, 0))


_GRID = NP // RBLK

_mmraw = pl.pallas_call(
    _mmraw_body,
    grid=(_GRID,),
    in_specs=[_row_spec(D_IN), _full_spec(D_IN, D_HID)],
    out_specs=_row_spec(D_HID),
    out_shape=jax.ShapeDtypeStruct((NP, D_HID), jnp.float32),
)

_scale = pl.pallas_call(
    _scale_body,
    grid=(_GRID,),
    in_specs=[_row_spec(B), _row_spec(B), _row_spec(D_HID)],
    out_specs=[_row_spec(D_HID), _row_spec(1)],
    out_shape=[jax.ShapeDtypeStruct((NP, D_HID), jnp.float32),
               jax.ShapeDtypeStruct((NP, 1), jnp.float32)],
)

_mm2 = pl.pallas_call(
    _mm2_body,
    grid=(_GRID,),
    in_specs=[_row_spec(D_HID), _row_spec(D_HID),
              _row_spec(1), _full_spec(1, D_HID), _full_spec(D_HID, D_HID)],
    out_specs=_row_spec(D_HID),
    out_shape=jax.ShapeDtypeStruct((NP, D_HID), jnp.float32),
)

_fin = pl.pallas_call(
    _fin_body,
    grid=(_GRID,),
    in_specs=[_row_spec(D_HID), _row_spec(D_HID),
              _row_spec(1), _full_spec(1, D_OUT)],
    out_specs=_row_spec(D_OUT),
    out_shape=jax.ShapeDtypeStruct((NP, D_OUT), jnp.float32),
)


@jax.jit
def _run(x, edge_index, W1, b1, W2, b2):
    srcp, dstp = _epad(edge_index[0].reshape(ROWS0, B),
                       edge_index[1].reshape(ROWS0, B))
    xp = jnp.pad(x, ((0, NP - N), (0, 0)))

    deg = _deg_kernel(dstp)
    xw1 = _mmraw(xp, W1)
    y1, dis = _scale(deg[0], deg[1], xw1)
    p = _agg_hid(srcp, dstp, y1)
    W2p = jnp.pad(W2, ((0, 0), (0, D_HID - D_OUT)))
    y2 = _mm2(p[0], p[1], dis, b1.reshape(1, D_HID), W2p)
    q = _agg_hid(srcp, dstp, y2)
    out = _fin(q[0], q[1], dis, b2.reshape(1, D_OUT))
    return out[:N]


def kernel(x, edge_index, W1, b1, W2, b2):
    return _run(x, edge_index, W1, b1, W2, b2)
